# baseline (speedup 1.0000x reference)
import jax
import jax.numpy as jnp
from jax import lax
from jax.experimental import pallas as pl
from jax.experimental.pallas import tpu as pltpu
from jax.experimental.pallas import tpu_sc as plsc

B, W, L = 1024, 20, 16
NUM_CHARS = 256
EMBED_DIM = 128
D3 = EMBED_DIM // 3
PAIRS = D3 // 2
TBL_WORDS = 5408
NC, NS = 2, 16
NW = NC * NS
WORDS = B * W
WPT = WORDS // NW
BPT = WPT // W
GROUPS = WPT // 16


def _sc_bow(table_hbm, ids_hbm, out_hbm, table_v, ids_v, out_v):
    wid = lax.axis_index("s") * NC + lax.axis_index("c")
    pltpu.sync_copy(table_hbm, table_v)
    pltpu.sync_copy(ids_hbm.at[wid], ids_v)

    iota = lax.iota(jnp.int32, 16)
    zeros = jnp.zeros((16,), jnp.float32)
    iota16 = iota * L

    def pair(fid, k):
        w = plsc.load_gather(table_v, [fid + k])
        return plsc.bitcast(w, jnp.bfloat16)

    def emit(vals, r0, r1, off, k):
        lo, hi = plsc.unpack(
            vals, format=plsc.PackFormat.INTERLEAVED,
            preferred_element_type=jnp.float32,
        )
        c = off + 2 * k
        plsc.store_scatter(out_v, [r0, r1, jnp.full((16,), c, jnp.int32)], lo)
        plsc.store_scatter(out_v, [r0, r1, jnp.full((16,), c + 1, jnp.int32)], hi)

    @plsc.parallel_loop(0, GROUPS, unroll=2)
    def group(g):
        r0 = g * 2 + (iota >> 3)
        r1 = iota & 7
        fid = [
            plsc.load_gather(ids_v, [iota16 + (g * (16 * L) + l)]) * PAIRS
            for l in range(L)
        ]
        for k in range(PAIRS):
            emit(pair(fid[0], k), r0, r1, 0, k)
            acc_a = pair(fid[1], k)
            for l in range(2, 8):
                acc_a = acc_a + pair(fid[l], k)
            acc_b = pair(fid[8], k)
            for l in range(9, L - 1):
                acc_b = acc_b + pair(fid[l], k)
            emit(acc_a + acc_b, r0, r1, D3, k)
            emit(pair(fid[L - 1], k), r0, r1, 2 * D3, k)
        zc = jnp.full((16,), 3 * D3, jnp.int32)
        plsc.store_scatter(out_v, [r0, r1, zc], zeros)
        plsc.store_scatter(out_v, [r0, r1, zc + 1], zeros)

    pltpu.sync_copy(out_v, out_hbm.at[pl.ds(wid * (WPT // 8), WPT // 8)])


@jax.jit
def kernel(word_ids, W_embed):
    tbl = jnp.pad(W_embed.T, ((0, 1), (0, 0)))
    lo = lax.bitcast_convert_type(tbl[:, 0::2].astype(jnp.bfloat16), jnp.uint16)
    hi = lax.bitcast_convert_type(tbl[:, 1::2].astype(jnp.bfloat16), jnp.uint16)
    packed = (hi.astype(jnp.int32) << 16) | lo.astype(jnp.int32)
    table = jnp.zeros((TBL_WORDS,), jnp.int32)
    table = lax.dynamic_update_slice(table, packed.reshape(-1), (0,))
    ids = word_ids.reshape(NW, WPT * L)

    mesh = plsc.VectorSubcoreMesh(
        core_axis_name="c", subcore_axis_name="s", num_cores=NC, num_subcores=NS
    )
    out = pl.kernel(
        _sc_bow,
        out_type=jax.ShapeDtypeStruct((WORDS // 8, 8, EMBED_DIM), jnp.float32),
        mesh=mesh,
        scratch_types=[
            pltpu.VMEM((TBL_WORDS,), jnp.int32),
            pltpu.VMEM((WPT * L,), jnp.int32),
            pltpu.VMEM((WPT // 8, 8, EMBED_DIM), jnp.float32),
        ],
        compiler_params=pltpu.CompilerParams(needs_layout_passes=False),
    )(table, ids)

    return out.reshape(B, W, EMBED_DIM)


# speedup vs baseline: 1.0843x; 1.0843x over previous
import jax
import jax.numpy as jnp
from jax import lax
from jax.experimental import pallas as pl
from jax.experimental.pallas import tpu as pltpu
from jax.experimental.pallas import tpu_sc as plsc

B, W, L = 1024, 20, 16
NUM_CHARS = 256
EMBED_DIM = 128
D3 = EMBED_DIM // 3
PAIRS = D3 // 2
TBL_WORDS = 5408
NC, NS = 2, 16
NW = NC * NS
WORDS = B * W
WPT = WORDS // NW
BPT = WPT // W
GROUPS = WPT // 16


def _sc_bow(table_hbm, ids_hbm, out_hbm, table_v, ids_v, out_v):
    wid = lax.axis_index("s") * NC + lax.axis_index("c")
    pltpu.sync_copy(table_hbm, table_v)
    pltpu.sync_copy(ids_hbm.at[pl.ds(wid * 10, 10)], ids_v)

    iota = lax.iota(jnp.int32, 16)
    zeros = jnp.zeros((16,), jnp.float32)
    iota16 = iota * L

    def pair(fid, k):
        w = plsc.load_gather(table_v, [fid + k])
        return plsc.bitcast(w, jnp.bfloat16)

    def emit(vals, r0, r1, off, k):
        lo, hi = plsc.unpack(
            vals, format=plsc.PackFormat.INTERLEAVED,
            preferred_element_type=jnp.float32,
        )
        c = off + 2 * k
        plsc.store_scatter(out_v, [r0, r1, jnp.full((16,), c, jnp.int32)], lo)
        plsc.store_scatter(out_v, [r0, r1, jnp.full((16,), c + 1, jnp.int32)], hi)

    @plsc.parallel_loop(0, GROUPS)
    def group(g):
        r0 = g * 2 + (iota >> 3)
        r1 = iota & 7
        ib = (g & 3) * 2 + (iota >> 3)
        ia = jnp.full((16,), g >> 2, jnp.int32)
        ic0 = (iota & 7) * L
        fid = [
            plsc.load_gather(ids_v, [ia, ib, ic0 + l]) * PAIRS
            for l in range(L)
        ]
        for k in range(PAIRS):
            emit(pair(fid[0], k), r0, r1, 0, k)
            acc_a = pair(fid[1], k)
            for l in range(2, 8):
                acc_a = acc_a + pair(fid[l], k)
            acc_b = pair(fid[8], k)
            for l in range(9, L - 1):
                acc_b = acc_b + pair(fid[l], k)
            emit(acc_a + acc_b, r0, r1, D3, k)
            emit(pair(fid[L - 1], k), r0, r1, 2 * D3, k)
        zc = jnp.full((16,), 3 * D3, jnp.int32)
        plsc.store_scatter(out_v, [r0, r1, zc], zeros)
        plsc.store_scatter(out_v, [r0, r1, zc + 1], zeros)

    pltpu.sync_copy(out_v, out_hbm.at[pl.ds(wid * (WPT // 8), WPT // 8)])


@jax.jit
def kernel(word_ids, W_embed):
    tbl = jnp.pad(W_embed.T, ((0, 1), (0, 0)))
    lo = lax.bitcast_convert_type(tbl[:, 0::2].astype(jnp.bfloat16), jnp.uint16)
    hi = lax.bitcast_convert_type(tbl[:, 1::2].astype(jnp.bfloat16), jnp.uint16)
    packed = (hi.astype(jnp.int32) << 16) | lo.astype(jnp.int32)
    table = jnp.zeros((TBL_WORDS,), jnp.int32)
    table = lax.dynamic_update_slice(table, packed.reshape(-1), (0,))

    mesh = plsc.VectorSubcoreMesh(
        core_axis_name="c", subcore_axis_name="s", num_cores=NC, num_subcores=NS
    )
    out = pl.kernel(
        _sc_bow,
        out_type=jax.ShapeDtypeStruct((WORDS // 8, 8, EMBED_DIM), jnp.float32),
        mesh=mesh,
        scratch_types=[
            pltpu.VMEM((TBL_WORDS,), jnp.int32),
            pltpu.VMEM((10, 8, EMBED_DIM), jnp.int32),
            pltpu.VMEM((WPT // 8, 8, EMBED_DIM), jnp.float32),
        ],
        compiler_params=pltpu.CompilerParams(needs_layout_passes=False),
    )(table, word_ids.reshape(NW * 10, 8, EMBED_DIM))

    return out.reshape(B, W, EMBED_DIM)


# speedup vs baseline: 1.1155x; 1.0288x over previous
import jax
import jax.numpy as jnp
from jax import lax
from jax.experimental import pallas as pl
from jax.experimental.pallas import tpu as pltpu
from jax.experimental.pallas import tpu_sc as plsc

B, W, L = 1024, 20, 16
NUM_CHARS = 256
EMBED_DIM = 128
D3 = EMBED_DIM // 3
PAIRS = D3 // 2
TBL_WORDS = 5408
NC, NS = 2, 16
NW = NC * NS
WORDS = B * W
WPT = WORDS // NW
BPT = WPT // W
GROUPS = WPT // 16


def _sc_bow(table_hbm, ids_hbm, out_hbm, table_v, ids_v, out_v):
    wid = lax.axis_index("s") * NC + lax.axis_index("c")
    pltpu.sync_copy(table_hbm, table_v)
    pltpu.sync_copy(ids_hbm.at[wid], ids_v)

    iota = lax.iota(jnp.int32, 16)
    zeros = jnp.zeros((16,), jnp.float32)
    iota16 = iota * L

    def pair(fid, k):
        w = plsc.load_gather(table_v, [fid + k])
        return plsc.bitcast(w, jnp.bfloat16)

    def emit(vals, bl, owbase, off, k):
        lo, hi = plsc.unpack(
            vals, format=plsc.PackFormat.INTERLEAVED,
            preferred_element_type=jnp.float32,
        )
        plsc.store_scatter(out_v, [bl, owbase + (off + 2 * k)], lo)
        plsc.store_scatter(out_v, [bl, owbase + (off + 2 * k + 1)], hi)

    @plsc.parallel_loop(0, GROUPS)
    def group(g):
        widx = wid * WPT + g * 16 + iota
        bvec = (widx * 52429) >> 20
        bl = bvec - wid * BPT
        owbase = (widx - bvec * W) * EMBED_DIM
        fid = [
            plsc.load_gather(ids_v, [iota16 + (g * (16 * L) + l)]) * PAIRS
            for l in range(L)
        ]
        for k in range(PAIRS):
            emit(pair(fid[0], k), bl, owbase, 0, k)
            acc_a = pair(fid[1], k)
            for l in range(2, 8):
                acc_a = acc_a + pair(fid[l], k)
            acc_b = pair(fid[8], k)
            for l in range(9, L - 1):
                acc_b = acc_b + pair(fid[l], k)
            emit(acc_a + acc_b, bl, owbase, D3, k)
            emit(pair(fid[L - 1], k), bl, owbase, 2 * D3, k)
        plsc.store_scatter(out_v, [bl, owbase + 3 * D3], zeros)
        plsc.store_scatter(out_v, [bl, owbase + (3 * D3 + 1)], zeros)

    pltpu.sync_copy(out_v, out_hbm.at[pl.ds(wid * BPT, BPT)])


@jax.jit
def kernel(word_ids, W_embed):
    tbl = jnp.pad(W_embed.T, ((0, 1), (0, 0)))
    lo = lax.bitcast_convert_type(tbl[:, 0::2].astype(jnp.bfloat16), jnp.uint16)
    hi = lax.bitcast_convert_type(tbl[:, 1::2].astype(jnp.bfloat16), jnp.uint16)
    packed = (hi.astype(jnp.int32) << 16) | lo.astype(jnp.int32)
    table = jnp.zeros((TBL_WORDS,), jnp.int32)
    table = lax.dynamic_update_slice(table, packed.reshape(-1), (0,))
    ids = word_ids.reshape(NW, WPT * L)

    mesh = plsc.VectorSubcoreMesh(
        core_axis_name="c", subcore_axis_name="s", num_cores=NC, num_subcores=NS
    )
    out = pl.kernel(
        _sc_bow,
        out_type=jax.ShapeDtypeStruct((B, W * EMBED_DIM), jnp.float32),
        mesh=mesh,
        scratch_types=[
            pltpu.VMEM((TBL_WORDS,), jnp.int32),
            pltpu.VMEM((WPT * L,), jnp.int32),
            pltpu.VMEM((BPT, W * EMBED_DIM), jnp.float32),
        ],
        compiler_params=pltpu.CompilerParams(needs_layout_passes=False),
    )(table, ids)
    return out.reshape(B, W, EMBED_DIM)
